# baseline (speedup 1.0000x reference)
import functools

import jax
import jax.numpy as jnp
from jax import lax
from jax.experimental import pallas as pl
from jax.experimental.pallas import tpu as pltpu
from jax.experimental.pallas import tpu_sc as plsc

_N_USERS = 5000
_N_ITEMS = 5000
_N = _N_USERS + _N_ITEMS
_DIM = 128
_E = 320000

_NC = 2
_NS = 16
_NW = _NC * _NS
_CHUNK = 96
_NCH = 108
_EPAD = _NW * _NCH * _CHUNK
_ACC_ROWS = 10240
_L = 16
_SROWS = _ACC_ROWS // _NS

_WROWS = _NCH * _CHUNK // 128

_mesh = plsc.VectorSubcoreMesh(core_axis_name="c", subcore_axis_name="s")
_CP = pltpu.CompilerParams(needs_layout_passes=False)
_CPF = pltpu.CompilerParams(needs_layout_passes=False,
                            use_tc_tiling_on_sc=False)


@functools.partial(
    pl.kernel,
    out_type=[
        jax.ShapeDtypeStruct((_NW, _NCH, _CHUNK), jnp.float32),
        jax.ShapeDtypeStruct((_NC, _ACC_ROWS), jnp.float32),
    ],
    mesh=_mesh,
    compiler_params=_CP,
    scratch_types=[
        pltpu.VMEM((_NCH, 2, _CHUNK), jnp.int32),
        pltpu.VMEM((_NCH, _CHUNK), jnp.float32),
        pltpu.VMEM((_NCH, _CHUNK), jnp.float32),
        pltpu.VMEM((_SROWS,), jnp.float32),
        pltpu.VMEM_SHARED((_ACC_ROWS,), jnp.float32),
    ],
)
def _sums_kernel(idx_h, w_h, expw_h, sums_h, idx_v, w_v, e_v, z_v, sums_sh):
    c = lax.axis_index("c")
    s = lax.axis_index("s")
    wid = s * _NC + c

    @pl.loop(0, _SROWS // _L)
    def _(i):
        z_v[pl.ds(i * _L, _L)] = jnp.zeros((_L,), jnp.float32)

    pltpu.sync_copy(z_v, sums_sh.at[pl.ds(s * _SROWS, _SROWS)])
    pltpu.sync_copy(idx_h.at[wid], idx_v)
    pltpu.sync_copy(w_h.at[wid], w_v)

    @pl.loop(0, _NCH)
    def _(j):
        for k in range(_CHUNK // _L):
            sl = pl.ds(k * _L, _L)
            e_v[j, sl] = jnp.exp(w_v[j, sl])

    pltpu.sync_copy(e_v, expw_h.at[wid])
    plsc.subcore_barrier()

    @pl.loop(0, _NCH)
    def _(j):
        pltpu.sync_copy(e_v.at[j], sums_sh.at[idx_v.at[j, 1]], add=True)

    plsc.subcore_barrier()
    pltpu.sync_copy(sums_sh.at[pl.ds(s * _SROWS, _SROWS)],
                    sums_h.at[c, pl.ds(s * _SROWS, _SROWS)])


@functools.partial(
    pl.kernel,
    out_type=[
        jax.ShapeDtypeStruct((_NC, _ACC_ROWS, _DIM), jnp.float32),
    ],
    mesh=_mesh,
    compiler_params=_CPF,
    scratch_types=[
        pltpu.VMEM((4, 2, _CHUNK), jnp.int32),
        pltpu.VMEM((_WROWS, 128), jnp.float32),
        pltpu.VMEM((_CHUNK, _DIM // 2), jnp.int32),
        pltpu.VMEM((_CHUNK, _DIM // 2), jnp.int32),
        pltpu.VMEM((_CHUNK, _DIM), jnp.float32),
        pltpu.VMEM((_CHUNK, _DIM), jnp.float32),
        [pltpu.SemaphoreType.DMA] * 2,
        [pltpu.SemaphoreType.DMA] * 2,
        [pltpu.SemaphoreType.DMA] * 4,
        pltpu.VMEM_SHARED((_ACC_ROWS, _DIM), jnp.float32),
    ],
)
def _prop_kernel(idx_h, w_h, emb_h, part_h,
                 idx_v, w_v, gp0, gp1, gf0, gf1, gs, ss, isem, acc_sh):
    c = lax.axis_index("c")
    s = lax.axis_index("s")
    wid = s * _NC + c
    gp = (gp0, gp1)
    gf = (gf0, gf1)

    @pl.loop(0, _CHUNK)
    def _(e):
        for k in range(_DIM // _L):
            gf0[e, pl.ds(k * _L, _L)] = jnp.zeros((_L,), jnp.float32)

    for z in range(_SROWS // _CHUNK):
        pltpu.sync_copy(gf0, acc_sh.at[pl.ds(s * _SROWS + z * _CHUNK, _CHUNK)])
    _zrem = _SROWS - (_SROWS // _CHUNK) * _CHUNK
    pltpu.sync_copy(
        gf0.at[pl.ds(0, _zrem)],
        acc_sh.at[pl.ds(s * _SROWS + _SROWS - _zrem, _zrem)])

    pltpu.sync_copy(w_h.at[wid], w_v)
    for k in range(3):
        pltpu.async_copy(idx_h.at[wid, k], idx_v.at[k], isem[k])
    plsc.subcore_barrier()
    for k in range(2):
        pltpu.make_async_copy(idx_h.at[wid, k], idx_v.at[k], isem[k]).wait()
        pltpu.async_copy(emb_h.at[idx_v.at[k, 0]], gp[k], gs[k])

    @pl.loop(0, _NCH, step=4)
    def _(j4):
        for b in range(4):
            j = j4 + b
            b2 = b % 2
            pltpu.make_async_copy(
                emb_h.at[idx_v.at[b, 0]], gp[b2], gs[b2]).wait()

            @pl.loop(0, _CHUNK // _L)
            def _(k16):
                f = j * _CHUNK + k16 * _L
                w16 = w_v[f // 128, pl.ds(f % 128, _L)]
                for i in range(_L):
                    w = w16[i]
                    e = k16 * _L + i
                    for m in range(_DIM // (2 * _L)):
                        v = gp[b2][e, pl.ds(m * _L, _L)]
                        lo, hi = plsc.unpack(
                            plsc.bitcast(v, jnp.bfloat16),
                            format=plsc.PackFormat.INTERLEAVED)
                        gf[b2][e, pl.ds(2 * m * _L, _L)] = lo * w
                        gf[b2][e, pl.ds((2 * m + 1) * _L, _L)] = hi * w

            pltpu.async_copy(
                gf[b2], acc_sh.at[idx_v.at[b, 1]], ss[b2], add=True)

            @pl.when(j + 2 < _NCH)
            def _():
                pltpu.make_async_copy(
                    idx_h.at[wid, j + 2], idx_v.at[(b + 2) % 4],
                    isem[(b + 2) % 4]).wait()
                pltpu.async_copy(
                    emb_h.at[idx_v.at[(b + 2) % 4, 0]], gp[b2], gs[b2])

            @pl.when(j >= 1)
            def _():
                pltpu.make_async_copy(
                    gf[(b2 + 1) % 2], acc_sh.at[idx_v.at[(b + 3) % 4, 1]],
                    ss[(b2 + 1) % 2]).wait()

            @pl.when(j + 3 < _NCH)
            def _():
                pltpu.async_copy(idx_h.at[wid, j + 3],
                                 idx_v.at[(b + 3) % 4], isem[(b + 3) % 4])

    pltpu.make_async_copy(
        gf[(_NCH - 1) % 2], acc_sh.at[idx_v.at[(_NCH - 1) % 4, 1]],
        ss[(_NCH - 1) % 2]).wait()
    plsc.subcore_barrier()
    pltpu.sync_copy(acc_sh.at[pl.ds(s * _SROWS, _SROWS)],
                    part_h.at[c, pl.ds(s * _SROWS, _SROWS)])


def _combine_tc(sp_ref, p_ref, o_ref):
    sums = sp_ref[0, :_N] + sp_ref[1, :_N]
    inv = jnp.where(sums > 0.0, 1.0 / sums, 0.0)
    o_ref[...] = (p_ref[0, :_N, :] + p_ref[1, :_N, :]) * inv[:, None]


def _mean_tc(e0_ref, e1_ref, sp_ref, p_ref, o_ref):
    sums = sp_ref[0, :_N] + sp_ref[1, :_N]
    inv = jnp.where(sums > 0.0, 1.0 / sums, 0.0)
    emb2 = (p_ref[0, :_N, :] + p_ref[1, :_N, :]) * inv[:, None]
    o_ref[...] = (e0_ref[...] + e1_ref[...] + emb2) * (1.0 / 3.0)


def _pack_rows(x):
    xb = x.astype(jnp.bfloat16).reshape(_N, _DIM // 32, 2, _L)
    u = jax.lax.bitcast_convert_type(xb, jnp.uint16).astype(jnp.uint32)
    packed = u[:, :, 0, :] | (u[:, :, 1, :] << 16)
    return jax.lax.bitcast_convert_type(
        packed, jnp.int32).reshape(_N, _DIM // 2)


def kernel(edge_index, edge_weight, user_emb, item_emb):
    row = edge_index[0].astype(jnp.int32)
    col = edge_index[1].astype(jnp.int32)
    npad = _EPAD - _E
    pad_i = jnp.arange(npad, dtype=jnp.int32)
    row_p = jnp.concatenate([row, _N + pad_i % (_ACC_ROWS - _N)])
    col_p = jnp.concatenate([col, pad_i % _N])
    w_p = jnp.concatenate(
        [edge_weight.astype(jnp.float32),
         jnp.full((npad,), -jnp.inf, jnp.float32)]).reshape(_NW, _NCH, _CHUNK)
    idx_p = jnp.stack(
        [col_p.reshape(_NW, _NCH, _CHUNK), row_p.reshape(_NW, _NCH, _CHUNK)],
        axis=2)
    emb0 = jnp.concatenate([user_emb, item_emb], axis=0)

    expw, sums_part = _sums_kernel(idx_p, w_p)
    expw128 = expw.reshape(_NW, _WROWS, 128)
    (part1,) = _prop_kernel(idx_p, expw128, _pack_rows(emb0))
    emb1 = pl.pallas_call(
        _combine_tc,
        out_shape=jax.ShapeDtypeStruct((_N, _DIM), jnp.float32))(
            sums_part, part1)
    (part2,) = _prop_kernel(idx_p, expw128, _pack_rows(emb1))
    out = pl.pallas_call(
        _mean_tc,
        out_shape=jax.ShapeDtypeStruct((_N, _DIM), jnp.float32))(
            emb0, emb1, sums_part, part2)
    return out[:_N_USERS], out[_N_USERS:]


# speedup vs baseline: 1.8608x; 1.8608x over previous
import functools

import jax
import jax.numpy as jnp
from jax import lax
from jax.experimental import pallas as pl
from jax.experimental.pallas import tpu as pltpu
from jax.experimental.pallas import tpu_sc as plsc

_N_USERS = 5000
_N_ITEMS = 5000
_N = _N_USERS + _N_ITEMS
_DIM = 128
_E = 320000

_NC = 2
_NS = 16
_NW = _NC * _NS
_CHUNK = 96
_NCH = 108
_EPAD = _NW * _NCH * _CHUNK
_ACC_ROWS = 10240
_L = 16
_SROWS = _ACC_ROWS // _NS

_WROWS = _NCH * _CHUNK // 128

_mesh = plsc.VectorSubcoreMesh(core_axis_name="c", subcore_axis_name="s")
_CP = pltpu.CompilerParams(needs_layout_passes=False)
_CPF = pltpu.CompilerParams(needs_layout_passes=False,
                            use_tc_tiling_on_sc=False)


@functools.partial(
    pl.kernel,
    out_type=[
        jax.ShapeDtypeStruct((_NW, _NCH, _CHUNK), jnp.float32),
        jax.ShapeDtypeStruct((_NC, _ACC_ROWS), jnp.float32),
    ],
    mesh=_mesh,
    compiler_params=_CP,
    scratch_types=[
        pltpu.VMEM((_NCH, 2, _CHUNK), jnp.int32),
        pltpu.VMEM((_NCH, _CHUNK), jnp.float32),
        pltpu.VMEM((_NCH, _CHUNK), jnp.float32),
        pltpu.VMEM((_SROWS,), jnp.float32),
        pltpu.VMEM_SHARED((_ACC_ROWS,), jnp.float32),
    ],
)
def _sums_kernel(idx_h, w_h, expw_h, sums_h, idx_v, w_v, e_v, z_v, sums_sh):
    c = lax.axis_index("c")
    s = lax.axis_index("s")
    wid = s * _NC + c

    @pl.loop(0, _SROWS // _L)
    def _(i):
        z_v[pl.ds(i * _L, _L)] = jnp.zeros((_L,), jnp.float32)

    pltpu.sync_copy(z_v, sums_sh.at[pl.ds(s * _SROWS, _SROWS)])
    pltpu.sync_copy(idx_h.at[wid], idx_v)
    pltpu.sync_copy(w_h.at[wid], w_v)

    @pl.loop(0, _NCH)
    def _(j):
        for k in range(_CHUNK // _L):
            sl = pl.ds(k * _L, _L)
            e_v[j, sl] = jnp.exp(w_v[j, sl])

    pltpu.sync_copy(e_v, expw_h.at[wid])
    plsc.subcore_barrier()

    @pl.loop(0, _NCH)
    def _(j):
        pltpu.sync_copy(e_v.at[j], sums_sh.at[idx_v.at[j, 1]], add=True)

    plsc.subcore_barrier()
    pltpu.sync_copy(sums_sh.at[pl.ds(s * _SROWS, _SROWS)],
                    sums_h.at[c, pl.ds(s * _SROWS, _SROWS)])


@functools.partial(
    pl.kernel,
    out_type=[
        jax.ShapeDtypeStruct((_NC, _ACC_ROWS, _DIM), jnp.float32),
    ],
    mesh=_mesh,
    compiler_params=_CP,
    scratch_types=[
        pltpu.VMEM((4, 2, _CHUNK), jnp.int32),
        pltpu.VMEM((_WROWS, 128), jnp.float32),
        pltpu.VMEM((_CHUNK, _DIM), jnp.float32),
        pltpu.VMEM((_CHUNK, _DIM), jnp.float32),
        pltpu.VMEM((_CHUNK, _DIM), jnp.float32),
        [pltpu.SemaphoreType.DMA] * 3,
        [pltpu.SemaphoreType.DMA] * 3,
        [pltpu.SemaphoreType.DMA] * 4,
        pltpu.VMEM_SHARED((_ACC_ROWS, _DIM), jnp.float32),
    ],
)
def _prop_kernel(idx_h, w_h, emb_h, part_h,
                 idx_v, w_v, g0, g1, g2, gs, ss, isem, acc_sh):
    c = lax.axis_index("c")
    s = lax.axis_index("s")
    wid = s * _NC + c
    g = (g0, g1, g2)

    @pl.loop(0, _CHUNK)
    def _(e):
        for k in range(_DIM // _L):
            g0[e, pl.ds(k * _L, _L)] = jnp.zeros((_L,), jnp.float32)

    for z in range(_SROWS // _CHUNK):
        pltpu.sync_copy(g0, acc_sh.at[pl.ds(s * _SROWS + z * _CHUNK, _CHUNK)])
    _zrem = _SROWS - (_SROWS // _CHUNK) * _CHUNK
    pltpu.sync_copy(
        g0.at[pl.ds(0, _zrem)],
        acc_sh.at[pl.ds(s * _SROWS + _SROWS - _zrem, _zrem)])

    pltpu.sync_copy(w_h.at[wid], w_v)
    for k in range(3):
        pltpu.async_copy(idx_h.at[wid, k], idx_v.at[k], isem[k])
    plsc.subcore_barrier()
    for k in range(2):
        pltpu.make_async_copy(idx_h.at[wid, k], idx_v.at[k], isem[k]).wait()
        pltpu.async_copy(emb_h.at[idx_v.at[k, 0]], g[k], gs[k])

    @pl.loop(0, _NCH, step=12)
    def _(j12):
        for b in range(12):
            j = j12 + b
            b3 = b % 3
            b4 = b % 4
            buf = g[b3]
            pltpu.make_async_copy(
                emb_h.at[idx_v.at[b4, 0]], buf, gs[b3]).wait()

            @pl.loop(0, _CHUNK // _L)
            def _(k16):
                f = j * _CHUNK + k16 * _L
                w16 = w_v[f // 128, pl.ds(f % 128, _L)]
                for i in range(_L):
                    w = w16[i]
                    e = k16 * _L + i
                    for m in range(_DIM // _L):
                        sl = pl.ds(m * _L, _L)
                        buf[e, sl] = buf[e, sl] * w

            pltpu.async_copy(buf, acc_sh.at[idx_v.at[b4, 1]], ss[b3], add=True)

            br = (b3 + 2) % 3

            @pl.when(j + 2 < _NCH)
            def _():
                @pl.when(j >= 1)
                def _():
                    pltpu.make_async_copy(
                        g[br], acc_sh.at[idx_v.at[(b4 + 3) % 4, 1]],
                        ss[br]).wait()
                pltpu.make_async_copy(
                    idx_h.at[wid, j + 2], idx_v.at[(b4 + 2) % 4],
                    isem[(b4 + 2) % 4]).wait()
                pltpu.async_copy(
                    emb_h.at[idx_v.at[(b4 + 2) % 4, 0]], g[br], gs[br])

            @pl.when(j + 3 < _NCH)
            def _():
                pltpu.async_copy(idx_h.at[wid, j + 3],
                                 idx_v.at[(b4 + 3) % 4], isem[(b4 + 3) % 4])

    for b3 in range(3):
        pltpu.make_async_copy(
            g[b3], acc_sh.at[idx_v.at[(_NCH - 3 + b3) % 4, 1]],
            ss[b3]).wait()
    plsc.subcore_barrier()
    pltpu.sync_copy(acc_sh.at[pl.ds(s * _SROWS, _SROWS)],
                    part_h.at[c, pl.ds(s * _SROWS, _SROWS)])


def _combine_tc(sp_ref, p_ref, o_ref):
    sums = sp_ref[0, :_N] + sp_ref[1, :_N]
    inv = jnp.where(sums > 0.0, 1.0 / sums, 0.0)
    o_ref[...] = (p_ref[0, :_N, :] + p_ref[1, :_N, :]) * inv[:, None]


def _mean_tc(e0_ref, e1_ref, sp_ref, p_ref, o_ref):
    sums = sp_ref[0, :_N] + sp_ref[1, :_N]
    inv = jnp.where(sums > 0.0, 1.0 / sums, 0.0)
    emb2 = (p_ref[0, :_N, :] + p_ref[1, :_N, :]) * inv[:, None]
    o_ref[...] = (e0_ref[...] + e1_ref[...] + emb2) * (1.0 / 3.0)


def kernel(edge_index, edge_weight, user_emb, item_emb):
    row = edge_index[0].astype(jnp.int32)
    col = edge_index[1].astype(jnp.int32)
    npad = _EPAD - _E
    pad_i = jnp.arange(npad, dtype=jnp.int32)
    row_p = jnp.concatenate([row, _N + pad_i % (_ACC_ROWS - _N)])
    col_p = jnp.concatenate([col, pad_i % _N])
    w_p = jnp.concatenate(
        [edge_weight.astype(jnp.float32),
         jnp.full((npad,), -jnp.inf, jnp.float32)]).reshape(_NW, _NCH, _CHUNK)
    idx_p = jnp.stack(
        [col_p.reshape(_NW, _NCH, _CHUNK), row_p.reshape(_NW, _NCH, _CHUNK)],
        axis=2)
    emb0 = jnp.concatenate([user_emb, item_emb], axis=0)

    expw, sums_part = _sums_kernel(idx_p, w_p)
    expw128 = expw.reshape(_NW, _WROWS, 128)
    (part1,) = _prop_kernel(idx_p, expw128, emb0)
    emb1 = pl.pallas_call(
        _combine_tc,
        out_shape=jax.ShapeDtypeStruct((_N, _DIM), jnp.float32))(
            sums_part, part1)
    (part2,) = _prop_kernel(idx_p, expw128, emb1)
    out = pl.pallas_call(
        _mean_tc,
        out_shape=jax.ShapeDtypeStruct((_N, _DIM), jnp.float32))(
            emb0, emb1, sums_part, part2)
    return out[:_N_USERS], out[_N_USERS:]
